# baseline (speedup 1.0000x reference)
import functools

import jax
import jax.numpy as jnp
from jax import lax
from jax.experimental import pallas as pl
from jax.experimental.pallas import tpu as pltpu
from jax.experimental.pallas import tpu_sc as plsc

_BATCH = 16384
_DIM = 32
_NC = 2
_NS = 16
_NW = _NC * _NS
_BPW = _BATCH // _NW
_Q = 4
_NQ = _BPW // _Q

_mesh = plsc.VectorSubcoreMesh(core_axis_name="c", subcore_axis_name="s")


def _bpr_body(user_id, pos_id, neg_id, ut, it,
              pos_hbm, neg_hbm,
              u_idx, p_idx, n_idx,
              ubuf0, pbuf0, nbuf0,
              ubuf1, pbuf1, nbuf1,
              pos_v, neg_v, sem0, sem1):
    wid = lax.axis_index("s") * _NC + lax.axis_index("c")
    base = wid * _BPW

    pltpu.sync_copy(user_id.at[pl.ds(base, _BPW)], u_idx)
    pltpu.sync_copy(pos_id.at[pl.ds(base, _BPW)], p_idx)
    pltpu.sync_copy(neg_id.at[pl.ds(base, _BPW)], n_idx)
    lane = lax.iota(jnp.int32, 16)

    sets = ((ubuf0, pbuf0, nbuf0, sem0), (ubuf1, pbuf1, nbuf1, sem1))

    def fire(grp, sub, s):
        bufs = sets[s]
        for k, (tab, idx) in enumerate(((ut, u_idx), (it, p_idx),
                                        (it, n_idx))):
            iv = idx[pl.ds(pl.multiple_of(grp * 16, 16), 16)]
            for j in range(_Q):
                idv = iv[sub * _Q + j]
                blk = pl.multiple_of(idv & -128, 128)
                pltpu.async_copy(tab.at[:, pl.ds(blk, 128)],
                                 bufs[k].at[j], bufs[3])

    def drain(s):
        bufs = sets[s]
        for k in range(3):
            for j in range(_Q):
                pltpu.make_async_copy(ut.at[:, pl.ds(0, 128)],
                                      bufs[k].at[j], bufs[3]).wait()

    def extract(sub, s, lid_u, lid_p, lid_n, accp, accn):
        drain(s)
        bufs = sets[s]
        jvec = jnp.clip(lane - sub * _Q, 0, _Q - 1)
        qp = jnp.zeros((16,), jnp.float32)
        qn = jnp.zeros((16,), jnp.float32)
        for d in range(_DIM):
            dcol = jnp.full((16,), d, jnp.int32)
            du = plsc.load_gather(bufs[0], [jvec, dcol, lid_u])
            dp = plsc.load_gather(bufs[1], [jvec, dcol, lid_p])
            dn = plsc.load_gather(bufs[2], [jvec, dcol, lid_n])
            qp = qp + du * dp
            qn = qn + du * dn
        inlane = (lane >> 2) == sub
        return jnp.where(inlane, qp, accp), jnp.where(inlane, qn, accn)

    fire(0, 0, 0)

    def body(g, carry):
        gbase = pl.multiple_of(g * 16, 16)
        lid_u = u_idx[pl.ds(gbase, 16)] & 127
        lid_p = p_idx[pl.ds(gbase, 16)] & 127
        lid_n = n_idx[pl.ds(gbase, 16)] & 127
        accp = jnp.zeros((16,), jnp.float32)
        accn = jnp.zeros((16,), jnp.float32)
        for sub in range(4):
            if sub < 3:
                fire(g, sub + 1, (sub + 1) % 2)
            else:
                fire(jnp.minimum(g + 1, _BPW // 16 - 1), 0, 0)
            accp, accn = extract(sub, sub % 2, lid_u, lid_p, lid_n,
                                 accp, accn)
        pos_v[pl.ds(gbase, 16)] = accp
        neg_v[pl.ds(gbase, 16)] = accn
        return carry

    lax.fori_loop(0, _BPW // 16, body, 0)
    drain(0)

    pltpu.sync_copy(pos_v, pos_hbm.at[pl.ds(base, _BPW)])
    pltpu.sync_copy(neg_v, neg_hbm.at[pl.ds(base, _BPW)])


def _build(interpret=False):
    return pl.kernel(
        _bpr_body,
        out_type=(
            jax.ShapeDtypeStruct((_BATCH,), jnp.float32),
            jax.ShapeDtypeStruct((_BATCH,), jnp.float32),
        ),
        mesh=_mesh,
        compiler_params=pltpu.CompilerParams(needs_layout_passes=False),
        scratch_types=[
            pltpu.VMEM((_BPW,), jnp.int32),
            pltpu.VMEM((_BPW,), jnp.int32),
            pltpu.VMEM((_BPW,), jnp.int32),
            pltpu.VMEM((_Q, _DIM, 128), jnp.float32),
            pltpu.VMEM((_Q, _DIM, 128), jnp.float32),
            pltpu.VMEM((_Q, _DIM, 128), jnp.float32),
            pltpu.VMEM((_Q, _DIM, 128), jnp.float32),
            pltpu.VMEM((_Q, _DIM, 128), jnp.float32),
            pltpu.VMEM((_Q, _DIM, 128), jnp.float32),
            pltpu.VMEM((_BPW,), jnp.float32),
            pltpu.VMEM((_BPW,), jnp.float32),
            pltpu.SemaphoreType.DMA,
            pltpu.SemaphoreType.DMA,
        ],
        interpret=interpret,
    )


_bpr_sc = _build()


def kernel(user_id, pos_id, neg_id, user_table, item_table):
    return _bpr_sc(user_id, pos_id, neg_id, user_table.T, item_table.T)


# speedup vs baseline: 1.0267x; 1.0267x over previous
import functools

import jax
import jax.numpy as jnp
from jax import lax
from jax.experimental import pallas as pl
from jax.experimental.pallas import tpu as pltpu
from jax.experimental.pallas import tpu_sc as plsc

_BATCH = 16384
_DIM = 32
_NC = 2
_NS = 16
_NW = _NC * _NS
_BPW = _BATCH // _NW
_G = 8
_NGRP = _BPW // _G

_mesh = plsc.VectorSubcoreMesh(core_axis_name="c", subcore_axis_name="s")


def _bpr_body(user_id, pos_id, neg_id, ut, it,
              pos_hbm, neg_hbm,
              u_idx, p_idx, n_idx,
              ubuf, pbuf, nbuf, ubuf1, pbuf1, nbuf1,
              pos_v, neg_v, sem, sem1):
    wid = lax.axis_index("s") * _NC + lax.axis_index("c")
    base = wid * _BPW

    pltpu.sync_copy(user_id.at[pl.ds(base, _BPW)], u_idx)
    pltpu.sync_copy(pos_id.at[pl.ds(base, _BPW)], p_idx)
    pltpu.sync_copy(neg_id.at[pl.ds(base, _BPW)], n_idx)
    lane = lax.iota(jnp.int32, 16)

    sets = ((ubuf, pbuf, nbuf, sem), (ubuf1, pbuf1, nbuf1, sem1))

    def fire(grp, s, setid):
        h, pp = s // 2, s % 2
        bufs = sets[setid]
        for k, (tab, idx) in enumerate(((ut, u_idx), (it, p_idx),
                                        (it, n_idx))):
            iv = idx[pl.ds(pl.multiple_of(grp * 16, 16), 16)]
            for j in range(_G):
                idv = iv[h * _G + j]
                blk = pl.multiple_of(idv & -128, 128)
                pltpu.async_copy(
                    tab.at[pl.ds(16 * pp, 16), pl.ds(blk, 128)],
                    bufs[k].at[j], bufs[3])

    def drain(setid):
        bufs = sets[setid]
        for k in range(3):
            for j in range(_G):
                pltpu.make_async_copy(ut.at[pl.ds(0, 16), pl.ds(0, 128)],
                                      bufs[k].at[j], bufs[3]).wait()

    def extract(s, setid, lid_u, lid_p, lid_n, accp, accn):
        drain(setid)
        bufs = sets[setid]
        h = s // 2
        jvec = jnp.clip(lane - h * _G, 0, _G - 1)
        qp = jnp.zeros((16,), jnp.float32)
        qn = jnp.zeros((16,), jnp.float32)
        for d in range(16):
            dcol = jnp.full((16,), d, jnp.int32)
            du = plsc.load_gather(bufs[0], [jvec, dcol, lid_u])
            dp = plsc.load_gather(bufs[1], [jvec, dcol, lid_p])
            dn = plsc.load_gather(bufs[2], [jvec, dcol, lid_n])
            qp = qp + du * dp
            qn = qn + du * dn
        inlane = (lane >> 3) == h
        return (jnp.where(inlane, accp + qp, accp),
                jnp.where(inlane, accn + qn, accn))

    fire(0, 0, 0)

    def body(g, carry):
        gbase = pl.multiple_of(g * 16, 16)
        lid_u = u_idx[pl.ds(gbase, 16)] & 127
        lid_p = p_idx[pl.ds(gbase, 16)] & 127
        lid_n = n_idx[pl.ds(gbase, 16)] & 127
        accp = jnp.zeros((16,), jnp.float32)
        accn = jnp.zeros((16,), jnp.float32)
        for s in range(4):
            if s < 3:
                fire(g, s + 1, (s + 1) % 2)
            else:
                fire(jnp.minimum(g + 1, _BPW // 16 - 1), 0, 0)
            accp, accn = extract(s, s % 2, lid_u, lid_p, lid_n, accp, accn)
        pos_v[pl.ds(gbase, 16)] = accp
        neg_v[pl.ds(gbase, 16)] = accn
        return carry

    lax.fori_loop(0, _BPW // 16, body, 0)
    drain(0)

    pltpu.sync_copy(pos_v, pos_hbm.at[pl.ds(base, _BPW)])
    pltpu.sync_copy(neg_v, neg_hbm.at[pl.ds(base, _BPW)])


def _build(interpret=False):
    return pl.kernel(
        _bpr_body,
        out_type=(
            jax.ShapeDtypeStruct((_BATCH,), jnp.float32),
            jax.ShapeDtypeStruct((_BATCH,), jnp.float32),
        ),
        mesh=_mesh,
        compiler_params=pltpu.CompilerParams(needs_layout_passes=False),
        scratch_types=[
            pltpu.VMEM((_BPW,), jnp.int32),
            pltpu.VMEM((_BPW,), jnp.int32),
            pltpu.VMEM((_BPW,), jnp.int32),
            pltpu.VMEM((_G, 16, 128), jnp.float32),
            pltpu.VMEM((_G, 16, 128), jnp.float32),
            pltpu.VMEM((_G, 16, 128), jnp.float32),
            pltpu.VMEM((_G, 16, 128), jnp.float32),
            pltpu.VMEM((_G, 16, 128), jnp.float32),
            pltpu.VMEM((_G, 16, 128), jnp.float32),
            pltpu.VMEM((_BPW,), jnp.float32),
            pltpu.VMEM((_BPW,), jnp.float32),
            pltpu.SemaphoreType.DMA,
            pltpu.SemaphoreType.DMA,
        ],
        interpret=interpret,
    )


_bpr_sc = _build()


def kernel(user_id, pos_id, neg_id, user_table, item_table):
    return _bpr_sc(user_id, pos_id, neg_id, user_table.T, item_table.T)


# speedup vs baseline: 1.0772x; 1.0492x over previous
import functools

import jax
import jax.numpy as jnp
from jax import lax
from jax.experimental import pallas as pl
from jax.experimental.pallas import tpu as pltpu
from jax.experimental.pallas import tpu_sc as plsc

_BATCH = 16384
_DIM = 32
_NC = 2
_NS = 16
_NW = _NC * _NS
_BPW = _BATCH // _NW
_G = 8
_NGRP = _BPW // _G

_mesh = plsc.VectorSubcoreMesh(core_axis_name="c", subcore_axis_name="s")


def _bpr_body(user_id, pos_id, neg_id, ut, it,
              pos_hbm, neg_hbm,
              u_idx, p_idx, n_idx,
              ubuf, pbuf, nbuf,
              pos_v, neg_v, sem):
    wid = lax.axis_index("s") * _NC + lax.axis_index("c")
    base = wid * _BPW

    pltpu.sync_copy(user_id.at[pl.ds(base, _BPW)], u_idx)
    pltpu.sync_copy(pos_id.at[pl.ds(base, _BPW)], p_idx)
    pltpu.sync_copy(neg_id.at[pl.ds(base, _BPW)], n_idx)
    lane = lax.iota(jnp.int32, 16)

    def body(g, carry):
        gbase = pl.multiple_of(g * 16, 16)
        iv_u = u_idx[pl.ds(gbase, 16)]
        iv_p = p_idx[pl.ds(gbase, 16)]
        iv_n = n_idx[pl.ds(gbase, 16)]
        lid_u = iv_u & 127
        lid_p = iv_p & 127
        lid_n = iv_n & 127
        halves = []
        for h in range(2):
            copies = []
            for tab, buf, iv in ((ut, ubuf, iv_u), (it, pbuf, iv_p),
                                 (it, nbuf, iv_n)):
                for j in range(_G):
                    idv = iv[h * _G + j]
                    blk = pl.multiple_of(idv & -128, 128)
                    copies.append(
                        pltpu.async_copy(tab.at[:, pl.ds(blk, 128)],
                                         buf.at[j], sem))
            for cp in copies:
                cp.wait()
            jvec = jnp.clip(lane - h * _G, 0, _G - 1)
            accp = jnp.zeros((16,), jnp.float32)
            accn = jnp.zeros((16,), jnp.float32)
            for d in range(_DIM):
                dcol = jnp.full((16,), d, jnp.int32)
                du = plsc.load_gather(ubuf, [jvec, dcol, lid_u])
                dp = plsc.load_gather(pbuf, [jvec, dcol, lid_p])
                dn = plsc.load_gather(nbuf, [jvec, dcol, lid_n])
                accp = accp + du * dp
                accn = accn + du * dn
            halves.append((accp, accn))
        lo = lane < _G
        pos_v[pl.ds(gbase, 16)] = jnp.where(lo, halves[0][0], halves[1][0])
        neg_v[pl.ds(gbase, 16)] = jnp.where(lo, halves[0][1], halves[1][1])
        return carry

    lax.fori_loop(0, _BPW // 16, body, 0)

    pltpu.sync_copy(pos_v, pos_hbm.at[pl.ds(base, _BPW)])
    pltpu.sync_copy(neg_v, neg_hbm.at[pl.ds(base, _BPW)])


def _build(interpret=False):
    return pl.kernel(
        _bpr_body,
        out_type=(
            jax.ShapeDtypeStruct((_BATCH,), jnp.float32),
            jax.ShapeDtypeStruct((_BATCH,), jnp.float32),
        ),
        mesh=_mesh,
        compiler_params=pltpu.CompilerParams(needs_layout_passes=False),
        scratch_types=[
            pltpu.VMEM((_BPW,), jnp.int32),
            pltpu.VMEM((_BPW,), jnp.int32),
            pltpu.VMEM((_BPW,), jnp.int32),
            pltpu.VMEM((_G, _DIM, 128), jnp.float32),
            pltpu.VMEM((_G, _DIM, 128), jnp.float32),
            pltpu.VMEM((_G, _DIM, 128), jnp.float32),
            pltpu.VMEM((_BPW,), jnp.float32),
            pltpu.VMEM((_BPW,), jnp.float32),
            pltpu.SemaphoreType.DMA,
        ],
        interpret=interpret,
    )


_bpr_sc = _build()


def kernel(user_id, pos_id, neg_id, user_table, item_table):
    return _bpr_sc(user_id, pos_id, neg_id, user_table.T, item_table.T)
